# Initial kernel scaffold; baseline (speedup 1.0000x reference)
import jax
import jax.numpy as jnp
from jax.experimental import pallas as pl


def kernel(x, edge_index, W1, b1, W2, b2):
    raise NotImplementedError("write your pallas kernel here")


# speedup vs baseline: 8.4141x; 8.4141x over previous
import functools

import jax
import jax.numpy as jnp
from jax import lax
from jax.experimental import pallas as pl
from jax.experimental.pallas import tpu as pltpu
from jax.experimental.pallas import tpu_sc as plsc

NNODE = 10000
DIM = 128
COUT = 121
NEDGE = 320000

NCORE = 2
NSUB = 16
NWORK = NCORE * NSUB
NPAD = 10240
RPT = NPAD // NSUB
CH = 128
CHUNKS = 80
EPAD = NWORK * CHUNKS * CH
DEGW = 16
BR = 1024

_mesh = plsc.VectorSubcoreMesh(
    core_axis_name="c", subcore_axis_name="s", num_cores=NCORE, num_subcores=NSUB
)


@functools.partial(
    pl.kernel,
    out_type=jax.ShapeDtypeStruct((NCORE * NPAD, DIM), jnp.float32),
    mesh=_mesh,
    scratch_types=[
        pltpu.VMEM((CHUNKS, CH), jnp.int32),
        pltpu.VMEM((CH, DIM), jnp.float32),
        pltpu.VMEM_SHARED((NPAD, DIM), jnp.float32),
    ],
)
def _deg_kernel(col_hbm, ones_hbm, zeros_hbm, out_hbm, col_v, buf_v, acc_sh):
    c = lax.axis_index("c")
    s = lax.axis_index("s")
    wid = c * NSUB + s
    pltpu.sync_copy(col_hbm.at[wid], col_v)
    pltpu.sync_copy(zeros_hbm, buf_v)
    for k in range(RPT // CH):
        pltpu.sync_copy(buf_v, acc_sh.at[pl.ds(s * RPT + k * CH, CH)])
    plsc.subcore_barrier()
    pltpu.sync_copy(ones_hbm, buf_v)

    def body(j, carry):
        pltpu.sync_copy(buf_v, acc_sh.at[col_v.at[j]], add=True)
        return carry

    lax.fori_loop(0, CHUNKS, body, 0)
    plsc.subcore_barrier()
    for k in range(RPT // CH):
        pltpu.sync_copy(acc_sh.at[pl.ds(s * RPT + k * CH, CH)], buf_v)
        pltpu.sync_copy(buf_v, out_hbm.at[pl.ds(c * NPAD + s * RPT + k * CH, CH)])


@functools.partial(
    pl.kernel,
    out_type=jax.ShapeDtypeStruct((NCORE * NPAD, DIM), jnp.float32),
    mesh=_mesh,
    scratch_types=[
        pltpu.VMEM((CHUNKS, CH), jnp.int32),
        pltpu.VMEM((CHUNKS, CH), jnp.int32),
        pltpu.VMEM((CH, DIM), jnp.float32),
        pltpu.SemaphoreType.DMA,
        pltpu.VMEM_SHARED((NPAD, DIM), jnp.float32),
    ],
)
def _prop_kernel(row_hbm, col_hbm, hs_hbm, zeros_hbm, out_hbm,
                 row_v, col_v, buf_v, sem, acc_sh):
    c = lax.axis_index("c")
    s = lax.axis_index("s")
    wid = c * NSUB + s
    pltpu.sync_copy(row_hbm.at[wid], row_v)
    pltpu.sync_copy(col_hbm.at[wid], col_v)
    pltpu.sync_copy(zeros_hbm, buf_v)
    for k in range(RPT // CH):
        pltpu.sync_copy(buf_v, acc_sh.at[pl.ds(s * RPT + k * CH, CH)])
    plsc.subcore_barrier()

    def body(j, carry):
        pltpu.async_copy(hs_hbm.at[row_v.at[j]], buf_v, sem).wait()
        pltpu.sync_copy(buf_v, acc_sh.at[col_v.at[j]], add=True)
        return carry

    lax.fori_loop(0, CHUNKS, body, 0)
    plsc.subcore_barrier()
    for k in range(RPT // CH):
        pltpu.sync_copy(acc_sh.at[pl.ds(s * RPT + k * CH, CH)], buf_v)
        pltpu.sync_copy(buf_v, out_hbm.at[pl.ds(c * NPAD + s * RPT + k * CH, CH)])


def _tc1_body(x_ref, w_ref, d0_ref, d1_ref, hs_ref, dv_ref):
    deg = d0_ref[:, :1] + d1_ref[:, :1] + 1.0
    dinv = lax.rsqrt(deg)
    h = jnp.dot(x_ref[...], w_ref[...], preferred_element_type=jnp.float32)
    hs_ref[...] = h * dinv
    dv_ref[...] = jnp.broadcast_to(dinv, (BR, DEGW))


_tc1 = pl.pallas_call(
    _tc1_body,
    grid=(NPAD // BR,),
    in_specs=[
        pl.BlockSpec((BR, DIM), lambda i: (i, 0)),
        pl.BlockSpec((DIM, DIM), lambda i: (0, 0)),
        pl.BlockSpec((BR, DIM), lambda i: (i, 0)),
        pl.BlockSpec((BR, DIM), lambda i: (i, 0)),
    ],
    out_specs=[
        pl.BlockSpec((BR, DIM), lambda i: (i, 0)),
        pl.BlockSpec((BR, DEGW), lambda i: (i, 0)),
    ],
    out_shape=[
        jax.ShapeDtypeStruct((NPAD, DIM), jnp.float32),
        jax.ShapeDtypeStruct((NPAD, DEGW), jnp.float32),
    ],
)


def _tc2_body(s0_ref, s1_ref, hs_ref, dv_ref, b1_ref, w2_ref, out_ref):
    dinv = dv_ref[:, :1]
    pre = (s0_ref[...] + s1_ref[...] + hs_ref[...]) * dinv + b1_ref[...]
    h1 = jnp.maximum(pre, 0.0)
    h2 = jnp.dot(h1, w2_ref[...], preferred_element_type=jnp.float32)
    out_ref[...] = h2 * dinv


_tc2 = pl.pallas_call(
    _tc2_body,
    grid=(NPAD // BR,),
    in_specs=[
        pl.BlockSpec((BR, DIM), lambda i: (i, 0)),
        pl.BlockSpec((BR, DIM), lambda i: (i, 0)),
        pl.BlockSpec((BR, DIM), lambda i: (i, 0)),
        pl.BlockSpec((BR, DEGW), lambda i: (i, 0)),
        pl.BlockSpec((1, DIM), lambda i: (0, 0)),
        pl.BlockSpec((DIM, DIM), lambda i: (0, 0)),
    ],
    out_specs=pl.BlockSpec((BR, DIM), lambda i: (i, 0)),
    out_shape=jax.ShapeDtypeStruct((NPAD, DIM), jnp.float32),
)


def _tc3_body(s0_ref, s1_ref, h2s_ref, dv_ref, b2_ref, out_ref):
    dinv = dv_ref[:, :1]
    out_ref[...] = (s0_ref[...] + s1_ref[...] + h2s_ref[...]) * dinv + b2_ref[...]


_tc3 = pl.pallas_call(
    _tc3_body,
    grid=(NPAD // BR,),
    in_specs=[
        pl.BlockSpec((BR, DIM), lambda i: (i, 0)),
        pl.BlockSpec((BR, DIM), lambda i: (i, 0)),
        pl.BlockSpec((BR, DIM), lambda i: (i, 0)),
        pl.BlockSpec((BR, DEGW), lambda i: (i, 0)),
        pl.BlockSpec((1, DIM), lambda i: (0, 0)),
    ],
    out_specs=pl.BlockSpec((BR, DIM), lambda i: (i, 0)),
    out_shape=jax.ShapeDtypeStruct((NPAD, DIM), jnp.float32),
)


def kernel(x, edge_index, W1, b1, W2, b2):
    xpad = jnp.zeros((NPAD, DIM), jnp.float32).at[:NNODE].set(x)
    pad_e = EPAD - NEDGE
    fill = jnp.full((pad_e,), NPAD - 1, jnp.int32)
    rowp = jnp.concatenate([edge_index[0], fill]).reshape(NWORK, CHUNKS, CH)
    colp = jnp.concatenate([edge_index[1], fill]).reshape(NWORK, CHUNKS, CH)

    ones128 = jnp.ones((CH, DIM), jnp.float32)
    zrows = jnp.zeros((CH, DIM), jnp.float32)

    degp = _deg_kernel(colp, ones128, zrows)
    deg0, deg1 = degp[:NPAD], degp[NPAD:]

    W2p = jnp.zeros((DIM, DIM), jnp.float32).at[:, :COUT].set(W2)
    b1r = b1.reshape(1, DIM)
    b2p = jnp.zeros((1, DIM), jnp.float32).at[0, :COUT].set(b2)

    hs, dinvc = _tc1(xpad, W1, deg0, deg1)
    s1 = _prop_kernel(rowp, colp, hs, zrows)
    h2s = _tc2(s1[:NPAD], s1[NPAD:], hs, dinvc, b1r, W2p)
    s2 = _prop_kernel(rowp, colp, h2s, zrows)
    outf = _tc3(s2[:NPAD], s2[NPAD:], h2s, dinvc, b2p)
    return outf[:NNODE, :COUT]


# speedup vs baseline: 9.4378x; 1.1217x over previous
import functools

import jax
import jax.numpy as jnp
from jax import lax
from jax.experimental import pallas as pl
from jax.experimental.pallas import tpu as pltpu
from jax.experimental.pallas import tpu_sc as plsc

NNODE = 10000
DIM = 128
COUT = 121
NEDGE = 320000

NCORE = 2
NSUB = 16
NWORK = NCORE * NSUB
NPAD = 10240
RPT = NPAD // NSUB
CH = 128
CHUNKS = 80
EPAD = NWORK * CHUNKS * CH
DEGW = 16
BR = 1024
CA = 116
CB = 44
FASTC = 0

_mesh = plsc.VectorSubcoreMesh(
    core_axis_name="c", subcore_axis_name="s", num_cores=NCORE, num_subcores=NSUB
)


@functools.partial(
    pl.kernel,
    out_type=jax.ShapeDtypeStruct((NCORE * NPAD, DIM), jnp.float32),
    mesh=_mesh,
    scratch_types=[
        pltpu.VMEM((CHUNKS, CH), jnp.int32),
        pltpu.VMEM((CH, DIM), jnp.float32),
        pltpu.VMEM_SHARED((NPAD, DIM), jnp.float32),
    ],
)
def _deg_kernel(col_hbm, ones_hbm, zeros_hbm, out_hbm, col_v, buf_v, acc_sh):
    c = lax.axis_index("c")
    s = lax.axis_index("s")
    wid = c * NSUB + s
    pltpu.sync_copy(col_hbm.at[wid], col_v)
    pltpu.sync_copy(zeros_hbm, buf_v)
    for k in range(RPT // CH):
        pltpu.sync_copy(buf_v, acc_sh.at[pl.ds(s * RPT + k * CH, CH)])
    plsc.subcore_barrier()
    pltpu.sync_copy(ones_hbm, buf_v)

    def body(j, carry):
        pltpu.sync_copy(buf_v, acc_sh.at[col_v.at[j]], add=True)
        return carry

    lax.fori_loop(0, CHUNKS, body, 0)
    plsc.subcore_barrier()
    for k in range(RPT // CH):
        pltpu.sync_copy(acc_sh.at[pl.ds(s * RPT + k * CH, CH)], buf_v)
        pltpu.sync_copy(buf_v, out_hbm.at[pl.ds(c * NPAD + s * RPT + k * CH, CH)])


@functools.partial(
    pl.kernel,
    out_type=jax.ShapeDtypeStruct((NCORE * NPAD, DIM), jnp.float32),
    mesh=_mesh,
    scratch_types=[
        pltpu.VMEM((CA, CH), jnp.int32),
        pltpu.VMEM((CA, CH), jnp.int32),
        pltpu.VMEM((CH, DIM), jnp.float32),
        pltpu.SemaphoreType.DMA,
        pltpu.VMEM_SHARED((NPAD, DIM), jnp.float32),
    ],
)
def _prop_kernel(row_hbm, col_hbm, hs_hbm, zeros_hbm, out_hbm,
                 row_v, col_v, buf_v, sem, acc_sh):
    c = lax.axis_index("c")
    s = lax.axis_index("s")
    wid = c * NSUB + s
    pltpu.sync_copy(row_hbm.at[wid], row_v)
    pltpu.sync_copy(col_hbm.at[wid], col_v)
    pltpu.sync_copy(zeros_hbm, buf_v)
    for k in range(RPT // CH):
        pltpu.sync_copy(buf_v, acc_sh.at[pl.ds(s * RPT + k * CH, CH)])
    plsc.subcore_barrier()

    def body(j, carry):
        pltpu.async_copy(hs_hbm.at[row_v.at[j]], buf_v, sem).wait()
        pltpu.sync_copy(buf_v, acc_sh.at[col_v.at[j]], add=True)
        return carry

    nch = jnp.where(c == FASTC, CA, CB)
    lax.fori_loop(0, nch, body, 0)
    plsc.subcore_barrier()
    for k in range(RPT // CH):
        pltpu.sync_copy(acc_sh.at[pl.ds(s * RPT + k * CH, CH)], buf_v)
        pltpu.sync_copy(buf_v, out_hbm.at[pl.ds(c * NPAD + s * RPT + k * CH, CH)])


def _tc1_body(x_ref, w_ref, d0_ref, d1_ref, hs_ref, dv_ref):
    deg = d0_ref[:, :1] + d1_ref[:, :1] + 1.0
    dinv = lax.rsqrt(deg)
    h = jnp.dot(x_ref[...], w_ref[...], preferred_element_type=jnp.float32)
    hs_ref[...] = h * dinv
    dv_ref[...] = jnp.broadcast_to(dinv, (BR, DEGW))


_tc1 = pl.pallas_call(
    _tc1_body,
    grid=(NPAD // BR,),
    in_specs=[
        pl.BlockSpec((BR, DIM), lambda i: (i, 0)),
        pl.BlockSpec((DIM, DIM), lambda i: (0, 0)),
        pl.BlockSpec((BR, DIM), lambda i: (i, 0)),
        pl.BlockSpec((BR, DIM), lambda i: (i, 0)),
    ],
    out_specs=[
        pl.BlockSpec((BR, DIM), lambda i: (i, 0)),
        pl.BlockSpec((BR, DEGW), lambda i: (i, 0)),
    ],
    out_shape=[
        jax.ShapeDtypeStruct((NPAD, DIM), jnp.float32),
        jax.ShapeDtypeStruct((NPAD, DEGW), jnp.float32),
    ],
)


def _tc2_body(s0_ref, s1_ref, hs_ref, dv_ref, b1_ref, w2_ref, out_ref):
    dinv = dv_ref[:, :1]
    pre = (s0_ref[...] + s1_ref[...] + hs_ref[...]) * dinv + b1_ref[...]
    h1 = jnp.maximum(pre, 0.0)
    h2 = jnp.dot(h1, w2_ref[...], preferred_element_type=jnp.float32)
    out_ref[...] = h2 * dinv


_tc2 = pl.pallas_call(
    _tc2_body,
    grid=(NPAD // BR,),
    in_specs=[
        pl.BlockSpec((BR, DIM), lambda i: (i, 0)),
        pl.BlockSpec((BR, DIM), lambda i: (i, 0)),
        pl.BlockSpec((BR, DIM), lambda i: (i, 0)),
        pl.BlockSpec((BR, DEGW), lambda i: (i, 0)),
        pl.BlockSpec((1, DIM), lambda i: (0, 0)),
        pl.BlockSpec((DIM, DIM), lambda i: (0, 0)),
    ],
    out_specs=pl.BlockSpec((BR, DIM), lambda i: (i, 0)),
    out_shape=jax.ShapeDtypeStruct((NPAD, DIM), jnp.float32),
)


def _tc3_body(s0_ref, s1_ref, h2s_ref, dv_ref, b2_ref, out_ref):
    dinv = dv_ref[:, :1]
    out_ref[...] = (s0_ref[...] + s1_ref[...] + h2s_ref[...]) * dinv + b2_ref[...]


_tc3 = pl.pallas_call(
    _tc3_body,
    grid=(NPAD // BR,),
    in_specs=[
        pl.BlockSpec((BR, DIM), lambda i: (i, 0)),
        pl.BlockSpec((BR, DIM), lambda i: (i, 0)),
        pl.BlockSpec((BR, DIM), lambda i: (i, 0)),
        pl.BlockSpec((BR, DEGW), lambda i: (i, 0)),
        pl.BlockSpec((1, DIM), lambda i: (0, 0)),
    ],
    out_specs=pl.BlockSpec((BR, DIM), lambda i: (i, 0)),
    out_shape=jax.ShapeDtypeStruct((NPAD, DIM), jnp.float32),
)


def kernel(x, edge_index, W1, b1, W2, b2):
    xpad = jnp.zeros((NPAD, DIM), jnp.float32).at[:NNODE].set(x)
    pad_e = EPAD - NEDGE
    fill = jnp.full((pad_e,), NPAD - 1, jnp.int32)
    rflat = jnp.concatenate([edge_index[0], fill])
    cflat = jnp.concatenate([edge_index[1], fill])
    colp = cflat.reshape(NWORK, CHUNKS, CH)

    def _asym(flat):
        split = NSUB * CA * CH
        big = flat[:split].reshape(NSUB, CA, CH)
        small = flat[split:].reshape(NSUB, CB, CH)
        small = jnp.concatenate(
            [small, jnp.full((NSUB, CA - CB, CH), NPAD - 1, jnp.int32)], axis=1)
        halves = [big, small] if FASTC == 0 else [small, big]
        return jnp.concatenate(halves, axis=0)

    rowp_a = _asym(rflat)
    colp_a = _asym(cflat)

    ones128 = jnp.ones((CH, DIM), jnp.float32)
    zrows = jnp.zeros((CH, DIM), jnp.float32)

    degp = _deg_kernel(colp, ones128, zrows)
    deg0, deg1 = degp[:NPAD], degp[NPAD:]

    W2p = jnp.zeros((DIM, DIM), jnp.float32).at[:, :COUT].set(W2)
    b1r = b1.reshape(1, DIM)
    b2p = jnp.zeros((1, DIM), jnp.float32).at[0, :COUT].set(b2)

    hs, dinvc = _tc1(xpad, W1, deg0, deg1)
    s1 = _prop_kernel(rowp_a, colp_a, hs, zrows)
    h2s = _tc2(s1[:NPAD], s1[NPAD:], hs, dinvc, b1r, W2p)
    s2 = _prop_kernel(rowp_a, colp_a, h2s, zrows)
    outf = _tc3(s2[:NPAD], s2[NPAD:], h2s, dinvc, b2p)
    return outf[:NNODE, :COUT]
